# baseline (speedup 1.0000x reference)
import functools

import jax
import jax.numpy as jnp
from jax import lax
from jax.experimental import pallas as pl
from jax.experimental.pallas import tpu as pltpu
from jax.experimental.pallas import tpu_sc as plsc

NV = 100000
ED = 48
B = 16384
N_CONT = 4
N_ID = 13
X_D = 17
OUT_D = N_CONT + N_ID * ED

NC = 2
NS = 16
NW = NC * NS
COLS_W = B // NW
CB = 64
N_CHUNK = COLS_W // CB
NRING = 4

_mesh = plsc.VectorSubcoreMesh(
    core_axis_name="c", subcore_axis_name="s", num_cores=NC, num_subcores=NS
)


@functools.partial(
    pl.kernel,
    out_type=jax.ShapeDtypeStruct((OUT_D, B), jnp.float32),
    mesh=_mesh,
    compiler_params=pltpu.CompilerParams(
        needs_layout_passes=False, use_tc_tiling_on_sc=False
    ),
    scratch_types=[
        pltpu.VMEM((X_D, COLS_W), jnp.float32),
        pltpu.VMEM((NRING, CB), jnp.int32),
        pltpu.VMEM((NRING, CB, ED), jnp.float32),
        pltpu.VMEM((2, OUT_D, CB), jnp.float32),
        [pltpu.SemaphoreType.DMA] * NRING,
        [pltpu.SemaphoreType.DMA] * 2,
    ],
)
def _card_emb(xt_hbm, emb_hbm, out_hbm, xt_v, idx_v, rows_v, outbuf_v, gsems, wsems):
    wid = lax.axis_index("s") * NC + lax.axis_index("c")
    bbase = wid * COLS_W

    lane = lax.iota(jnp.int32, 16)

    pltpu.sync_copy(xt_hbm.at[:, pl.ds(bbase, COLS_W)], xt_v)

    seg_rows = [
        [N_CONT + j * ED + m * 16 + lane for m in range(ED // 16)]
        for j in range(N_ID)
    ]

    def chunk_body(k, carry):
        p = lax.bitwise_and(k, 1)
        cb0 = k * CB
        obuf = outbuf_v.at[p]

        for par in range(2):
            @pl.when(jnp.logical_and(k >= 2, p == par))
            def _(par=par):
                pltpu.make_async_copy(
                    obuf, out_hbm.at[:, pl.ds(bbase, CB)], wsems[par]
                ).wait()

        def build_and_fire(j):
            slot = j % NRING
            for g in range(CB // 16):
                vals = xt_v[N_CONT + j, pl.ds(cb0 + g * 16, 16)]
                idx_v[slot, pl.ds(g * 16, 16)] = vals.astype(jnp.int32)
            pltpu.async_copy(
                emb_hbm.at[idx_v.at[slot]], rows_v.at[slot], gsems[slot]
            )

        for j in range(min(NRING - 1, N_ID)):
            build_and_fire(j)

        for c in range(N_CONT):
            for g in range(CB // 16):
                obuf[c, pl.ds(g * 16, 16)] = xt_v[c, pl.ds(cb0 + g * 16, 16)]

        for j in range(N_ID):
            if j + NRING - 1 < N_ID:
                build_and_fire(j + NRING - 1)
            slot = j % NRING
            pltpu.make_async_copy(
                emb_hbm.at[idx_v.at[slot]], rows_v.at[slot], gsems[slot]
            ).wait()
            rbuf = rows_v.at[slot]
            rows_jm = seg_rows[j]

            def copy_body(r, cc):
                colv = jnp.full((16,), r, jnp.int32)
                for m in range(ED // 16):
                    v = rbuf[r, pl.ds(m * 16, 16)]
                    plsc.store_scatter(obuf, [rows_jm[m], colv], v)
                return cc

            lax.fori_loop(0, CB, copy_body, 0)

        for par in range(2):
            @pl.when(p == par)
            def _(par=par):
                pltpu.async_copy(
                    obuf, out_hbm.at[:, pl.ds(bbase + cb0, CB)], wsems[par]
                )
        return carry

    lax.fori_loop(0, N_CHUNK, chunk_body, 0)

    for par in range(2):
        pltpu.make_async_copy(
            outbuf_v.at[par], out_hbm.at[:, pl.ds(bbase, CB)], wsems[par]
        ).wait()


def kernel(x, emb):
    out_t = _card_emb(x.T, emb)
    return out_t.T


# speedup vs baseline: 1.0398x; 1.0398x over previous
import functools

import jax
import jax.numpy as jnp
from jax import lax
from jax.experimental import pallas as pl
from jax.experimental.pallas import tpu as pltpu
from jax.experimental.pallas import tpu_sc as plsc

NV = 100000
ED = 48
B = 16384
N_CONT = 4
N_ID = 13
X_D = 17
OUT_D = N_CONT + N_ID * ED

NC = 2
NS = 16
NW = NC * NS
COLS_W = B // NW
CB = 64
N_CHUNK = COLS_W // CB
NRING = 4

_mesh = plsc.VectorSubcoreMesh(
    core_axis_name="c", subcore_axis_name="s", num_cores=NC, num_subcores=NS
)


@functools.partial(
    pl.kernel,
    out_type=jax.ShapeDtypeStruct((OUT_D, B), jnp.float32),
    mesh=_mesh,
    compiler_params=pltpu.CompilerParams(
        needs_layout_passes=False, use_tc_tiling_on_sc=False
    ),
    scratch_types=[
        pltpu.VMEM((X_D, COLS_W), jnp.float32),
        pltpu.VMEM((NRING, CB), jnp.int32),
        pltpu.VMEM((NRING, CB, ED), jnp.float32),
        pltpu.VMEM((2, OUT_D, CB), jnp.float32),
        [pltpu.SemaphoreType.DMA] * NRING,
        [pltpu.SemaphoreType.DMA] * 2,
    ],
)
def _card_emb(xt_hbm, emb_hbm, out_hbm, xt_v, idx_v, rows_v, outbuf_v, gsems, wsems):
    wid = lax.axis_index("s") * NC + lax.axis_index("c")
    bbase = wid * COLS_W

    lane = lax.iota(jnp.int32, 16)

    pltpu.sync_copy(xt_hbm.at[:, pl.ds(bbase, COLS_W)], xt_v)

    seg_rows = [
        [N_CONT + j * ED + m * 16 + lane for m in range(ED // 16)]
        for j in range(N_ID)
    ]

    def chunk_body(k, carry):
        p = lax.bitwise_and(k, 1)
        cb0 = k * CB
        obuf = outbuf_v.at[p]

        for par in range(2):
            @pl.when(jnp.logical_and(k >= 2, p == par))
            def _(par=par):
                pltpu.make_async_copy(
                    obuf, out_hbm.at[:, pl.ds(bbase, CB)], wsems[par]
                ).wait()

        def build_and_fire(j):
            slot = j % NRING
            for g in range(CB // 16):
                vals = xt_v[N_CONT + j, pl.ds(cb0 + g * 16, 16)]
                idx_v[slot, pl.ds(g * 16, 16)] = vals.astype(jnp.int32)
            pltpu.async_copy(
                emb_hbm.at[idx_v.at[slot]], rows_v.at[slot], gsems[slot]
            )

        for j in range(min(NRING - 1, N_ID)):
            build_and_fire(j)

        for c in range(N_CONT):
            for g in range(CB // 16):
                obuf[c, pl.ds(g * 16, 16)] = xt_v[c, pl.ds(cb0 + g * 16, 16)]

        for j in range(N_ID):
            if j + NRING - 1 < N_ID:
                build_and_fire(j + NRING - 1)
            slot = j % NRING
            pltpu.make_async_copy(
                emb_hbm.at[idx_v.at[slot]], rows_v.at[slot], gsems[slot]
            ).wait()
            rbuf = rows_v.at[slot]
            rows_jm = seg_rows[j]

            def copy_body(i, cc):
                r0 = i * 4
                vs = []
                for dr in range(4):
                    for m in range(ED // 16):
                        vs.append(rbuf[r0 + dr, pl.ds(m * 16, 16)])
                for dr in range(4):
                    colv = jnp.full((16,), r0 + dr, jnp.int32)
                    for m in range(ED // 16):
                        plsc.store_scatter(
                            obuf, [rows_jm[m], colv], vs[dr * 3 + m]
                        )
                return cc

            lax.fori_loop(0, CB // 4, copy_body, 0)

        for par in range(2):
            @pl.when(p == par)
            def _(par=par):
                pltpu.async_copy(
                    obuf, out_hbm.at[:, pl.ds(bbase + cb0, CB)], wsems[par]
                )
        return carry

    lax.fori_loop(0, N_CHUNK, chunk_body, 0)

    for par in range(2):
        pltpu.make_async_copy(
            outbuf_v.at[par], out_hbm.at[:, pl.ds(bbase, CB)], wsems[par]
        ).wait()


def kernel(x, emb):
    out_t = _card_emb(x.T, emb)
    return out_t.T


# speedup vs baseline: 1.4403x; 1.3853x over previous
import functools

import jax
import jax.numpy as jnp
from jax import lax
from jax.experimental import pallas as pl
from jax.experimental.pallas import tpu as pltpu
from jax.experimental.pallas import tpu_sc as plsc

NV = 100000
ED = 48
B = 16384
N_CONT = 4
N_ID = 13
X_D = 17
OUT_D = N_CONT + N_ID * ED

NC = 2
NS = 16
NW = NC * NS
COLS_W = B // NW
CB = 64
N_CHUNK = COLS_W // CB
NRING = 2

_mesh = plsc.VectorSubcoreMesh(
    core_axis_name="c", subcore_axis_name="s", num_cores=NC, num_subcores=NS
)


@functools.partial(
    pl.kernel,
    out_type=jax.ShapeDtypeStruct((OUT_D, B), jnp.float32),
    mesh=_mesh,
    compiler_params=pltpu.CompilerParams(
        needs_layout_passes=False, use_tc_tiling_on_sc=False
    ),
    scratch_types=[
        pltpu.VMEM((X_D, COLS_W), jnp.float32),
        pltpu.VMEM((NRING, CB), jnp.int32),
        pltpu.VMEM((NRING, CB, ED), jnp.float32),
        pltpu.VMEM((2, OUT_D, CB), jnp.float32),
        [pltpu.SemaphoreType.DMA] * NRING,
        [pltpu.SemaphoreType.DMA] * 2,
    ],
)
def _card_emb(xt_hbm, emb_hbm, out_hbm, xt_v, idx_v, rows_v, outbuf_v, gsems, wsems):
    wid = lax.axis_index("s") * NC + lax.axis_index("c")
    bbase = wid * COLS_W

    lane = lax.iota(jnp.int32, 16)

    pltpu.sync_copy(xt_hbm.at[:, pl.ds(bbase, COLS_W)], xt_v)


    def chunk_body(k, carry):
        p = lax.bitwise_and(k, 1)
        cb0 = k * CB
        obuf = outbuf_v.at[p]

        for par in range(2):
            @pl.when(jnp.logical_and(k >= 2, p == par))
            def _(par=par):
                pltpu.make_async_copy(
                    obuf, out_hbm.at[:, pl.ds(bbase, CB)], wsems[par]
                ).wait()

        def build_and_fire(j):
            slot = j % NRING
            for g in range(CB // 16):
                vals = xt_v[N_CONT + j, pl.ds(cb0 + g * 16, 16)]
                idx_v[slot, pl.ds(g * 16, 16)] = vals.astype(jnp.int32)
            pltpu.async_copy(
                emb_hbm.at[idx_v.at[slot]], rows_v.at[slot], gsems[slot]
            )

        for j in range(min(NRING - 1, N_ID)):
            build_and_fire(j)

        for c in range(N_CONT):
            for g in range(CB // 16):
                obuf[c, pl.ds(g * 16, 16)] = xt_v[c, pl.ds(cb0 + g * 16, 16)]

        for j in range(N_ID):
            if j + NRING - 1 < N_ID:
                build_and_fire(j + NRING - 1)
            slot = j % NRING
            pltpu.make_async_copy(
                emb_hbm.at[idx_v.at[slot]], rows_v.at[slot], gsems[slot]
            ).wait()
            rbuf = rows_v.at[slot]

            def copy_body(it, cc):
                half = lax.bitwise_and(it, 1) * 8
                ib = lax.shift_right_logical(it, 1)
                i = ib // (ED // 16)
                f0 = (ib % (ED // 16)) * 16
                colv = lane + i * 16
                for t in range(8):
                    dg = lax.bitwise_and(lane + half + t, 15) + f0
                    v = plsc.load_gather(rbuf, [colv, dg])
                    plsc.store_scatter(obuf, [N_CONT + j * ED + dg, colv], v)
                return cc

            lax.fori_loop(0, (CB // 16) * (ED // 16) * 2, copy_body, 0)

        for par in range(2):
            @pl.when(p == par)
            def _(par=par):
                pltpu.async_copy(
                    obuf, out_hbm.at[:, pl.ds(bbase + cb0, CB)], wsems[par]
                )
        return carry

    lax.fori_loop(0, N_CHUNK, chunk_body, 0)

    for par in range(2):
        pltpu.make_async_copy(
            outbuf_v.at[par], out_hbm.at[:, pl.ds(bbase, CB)], wsems[par]
        ).wait()


def kernel(x, emb):
    out_t = _card_emb(x.T, emb)
    return out_t.T
